# Initial kernel scaffold; baseline (speedup 1.0000x reference)
import jax
import jax.numpy as jnp
from jax.experimental import pallas as pl


def kernel(x, edge_index_r1, edge_index_r2, e1, e2, W_bl, W_bl1, b_bl1):
    raise NotImplementedError("write your pallas kernel here")


# speedup vs baseline: 5.6424x; 5.6424x over previous
import jax
import jax.numpy as jnp
from jax import lax
from jax.experimental import pallas as pl
from jax.experimental.pallas import tpu as pltpu
from jax.experimental.pallas import tpu_sc as plsc

N_NODES = 10000
N_EDGES = 160000
TE = 2 * N_EDGES
D_NODE = 768
D_OUT = 32
TWP = 128
NP1 = 10240

NC, NS = 2, 16
NW = NC * NS
EP = 163840
EW = EP // NW
CA = 128
NCHUNK_A = EW // CA
PER_WB = TE // NW
CB = 400
NCHUNK_B = PER_WB // CB
ROWS_SUB = NP1 // NS


def _sc_mesh():
    return plsc.VectorSubcoreMesh(core_axis_name="c", subcore_axis_name="s",
                                  num_cores=NC, num_subcores=NS)


def _tc1_body(x_ref, w_ref, o_ref):
    xw = jnp.dot(x_ref[...], w_ref[...], preferred_element_type=jnp.float32)
    o_ref[...] = jnp.concatenate(
        [xw, jnp.zeros((xw.shape[0], TWP - D_OUT), jnp.float32)], axis=1)


def _tc1(x, w):
    blk = 2000
    return pl.pallas_call(
        _tc1_body,
        grid=(N_NODES // blk,),
        in_specs=[
            pl.BlockSpec((blk, D_NODE), lambda i: (i, 0)),
            pl.BlockSpec((D_NODE, D_OUT), lambda i: (0, 0)),
        ],
        out_specs=pl.BlockSpec((blk, TWP), lambda i: (i, 0)),
        out_shape=jax.ShapeDtypeStruct((N_NODES, TWP), jnp.float32),
    )(x, w)


def _sca_body(xwp, e1d, dsti, part, tab_s, idx_v, ev, xg, rows, sem):
    c = lax.axis_index("c")
    s = lax.axis_index("s")
    wid = s * NC + c
    zero16 = jnp.zeros((16,), jnp.float32)
    lane = lax.iota(jnp.int32, 16)
    unit16 = (lane == 0).astype(jnp.float32)

    def _z1(i, carry):
        for k in range(TWP // 16):
            rows[i, pl.ds(k * 16, 16)] = zero16
        return carry
    lax.fori_loop(0, CA, _z1, 0)

    for z in range(ROWS_SUB // CA):
        pltpu.sync_copy(rows, tab_s.at[pl.ds(s * ROWS_SUB + z * CA, CA)])
    plsc.subcore_barrier()

    base0 = wid * EW

    def _edge(j, jj, acc):
        a0 = xg[j, pl.ds(0, 16)]
        b0 = ev[pl.ds(j * 32, 16)]
        a1 = xg[j, pl.ds(16, 16)]
        b1 = ev[pl.ds(j * 32 + 16, 16)]
        sj = jnp.sum(a0 * b0 + a1 * b1)
        return jnp.where(lane == jj, sj, acc)

    def _emit(j, ex_s):
        rows[j, pl.ds(0, 16)] = ev[pl.ds(j * 32, 16)] * ex_s
        rows[j, pl.ds(16, 16)] = ev[pl.ds(j * 32 + 16, 16)] * ex_s
        rows[j, pl.ds(32, 16)] = unit16 * ex_s

    def _chunk(g, carry):
        base = pl.multiple_of(base0 + g * CA, 8)
        pltpu.sync_copy(dsti.at[pl.ds(base, CA)], idx_v)
        pltpu.sync_copy(e1d.at[pl.ds(base * 32, CA * 32)], ev)
        pltpu.async_copy(xwp.at[idx_v], xg, sem).wait()

        def _grp(t, carry2):
            jb = t * 16
            acc = jnp.zeros((16,), jnp.float32)
            for jj in range(16):
                acc = _edge(jb + jj, jj, acc)
            ex = jnp.exp(acc)
            for jj in range(16):
                _emit(jb + jj, ex[jj])
            return carry2
        lax.fori_loop(0, CA // 16, _grp, 0)

        pltpu.sync_copy(rows, tab_s.at[idx_v], add=True)
        return carry
    lax.fori_loop(0, NCHUNK_A, _chunk, 0)

    plsc.subcore_barrier()
    pltpu.sync_copy(tab_s.at[pl.ds(s * ROWS_SUB, ROWS_SUB)],
                    part.at[pl.ds(c * NP1 + s * ROWS_SUB, ROWS_SUB)])


def _sca(xwp, e1d, dsti):
    return pl.kernel(
        _sca_body,
        out_type=jax.ShapeDtypeStruct((NC * NP1, TWP), jnp.float32),
        mesh=_sc_mesh(),
        compiler_params=pltpu.CompilerParams(needs_layout_passes=False),
        scratch_types=[
            pltpu.VMEM_SHARED((NP1, TWP), jnp.float32),
            pltpu.VMEM((CA,), jnp.int32),
            pltpu.VMEM((CA * D_OUT,), jnp.float32),
            pltpu.VMEM((CA, TWP), jnp.float32),
            pltpu.VMEM((CA, TWP), jnp.float32),
            pltpu.SemaphoreType.DMA,
        ],
    )(xwp, e1d, dsti)


def _tc2_body(p1_ref, p2_ref, o_ref):
    p1 = p1_ref[...]
    p2 = p2_ref[...]
    s1 = p1[0] + p1[1]
    s2 = p2[0] + p2[1]
    d1 = s1[:, D_OUT:D_OUT + 1]
    d2 = s2[:, D_OUT:D_OUT + 1]
    r1 = s1[:, 0:D_OUT] / jnp.where(d1 > 0.0, d1, 1.0)
    r2 = s2[:, 0:D_OUT] / jnp.where(d2 > 0.0, d2, 1.0)
    o_ref[...] = jnp.concatenate([r1, r2, r1, r2], axis=1)


def _tc2(part1, part2):
    blk = 2000
    return pl.pallas_call(
        _tc2_body,
        grid=(N_NODES // blk,),
        in_specs=[
            pl.BlockSpec((2, blk, TWP), lambda i: (0, i, 0)),
            pl.BlockSpec((2, blk, TWP), lambda i: (0, i, 0)),
        ],
        out_specs=pl.BlockSpec((blk, TWP), lambda i: (i, 0)),
        out_shape=jax.ShapeDtypeStruct((N_NODES, TWP), jnp.float32),
    )(part1, part2)


def _scb_body(relf, srci, dsti, s_out, d_out, idx_v, buf, sem):
    c = lax.axis_index("c")
    s = lax.axis_index("s")
    wid = s * NC + c
    base0 = wid * PER_WB

    def _chunk(g, carry):
        base = pl.multiple_of(base0 + g * CB, 8)
        pltpu.sync_copy(srci.at[pl.ds(base, CB)], idx_v)
        pltpu.async_copy(relf.at[idx_v], buf, sem).wait()
        pltpu.sync_copy(buf, s_out.at[pl.ds(base, CB)])
        pltpu.sync_copy(dsti.at[pl.ds(base, CB)], idx_v)
        pltpu.async_copy(relf.at[idx_v], buf, sem).wait()
        pltpu.sync_copy(buf, d_out.at[pl.ds(base, CB)])
        return carry
    lax.fori_loop(0, NCHUNK_B, _chunk, 0)


def _scb(relf, srci, dsti):
    sds = jax.ShapeDtypeStruct((TE, TWP), jnp.float32)
    return pl.kernel(
        _scb_body,
        out_type=(sds, sds),
        mesh=_sc_mesh(),
        compiler_params=pltpu.CompilerParams(use_tc_tiling_on_sc=True),
        scratch_types=[
            pltpu.VMEM((CB,), jnp.int32),
            pltpu.VMEM((CB, TWP), jnp.float32),
            pltpu.SemaphoreType.DMA,
        ],
    )(relf, srci, dsti)


def _tc3_body(s_ref, d_ref, e_ref, wf_ref, sm_ref, b_ref, o_ref):
    sb = s_ref[...][:, 0:2 * D_OUT]
    t = jnp.dot(sb, wf_ref[...],
                preferred_element_type=jnp.float32)
    d2 = d_ref[...]
    drep = jnp.concatenate([d2] * 16, axis=1)
    p = t * drep
    nr = jnp.dot(p, sm_ref[...],
                 preferred_element_type=jnp.float32) + b_ref[...]
    eb = e_ref[...]
    o_ref[...] = eb + jax.nn.sigmoid(nr) * eb


def _tc3(sg, dg, ecat, wf, sm, b):
    blk = 640
    k2 = D_OUT * 2 * D_OUT
    return pl.pallas_call(
        _tc3_body,
        grid=(TE // blk,),
        in_specs=[
            pl.BlockSpec((blk, TWP), lambda i: (i, 0)),
            pl.BlockSpec((blk, TWP), lambda i: (i, 0)),
            pl.BlockSpec((blk, D_OUT), lambda i: (i, 0)),
            pl.BlockSpec((2 * D_OUT, k2), lambda i: (0, 0)),
            pl.BlockSpec((k2, D_OUT), lambda i: (0, 0)),
            pl.BlockSpec((1, D_OUT), lambda i: (0, 0)),
        ],
        out_specs=pl.BlockSpec((blk, D_OUT), lambda i: (i, 0)),
        out_shape=jax.ShapeDtypeStruct((TE, D_OUT), jnp.float32),
    )(sg, dg, ecat, wf, sm, b)


def kernel(x, edge_index_r1, edge_index_r2, e1, e2, W_bl, W_bl1, b_bl1):
    xwp = _tc1(x, W_bl[0])
    xwp_p = jnp.concatenate(
        [xwp, jnp.zeros((NP1 - N_NODES, TWP), jnp.float32)], axis=0)
    npad = EP - N_EDGES
    epad = jnp.zeros((npad * D_OUT,), jnp.float32)
    ipad = jnp.full((npad,), N_NODES, jnp.int32)
    e1p = jnp.concatenate([e1.reshape(-1), epad])
    e2p = jnp.concatenate([e2.reshape(-1), epad])
    d1p = jnp.concatenate([edge_index_r1[1], ipad])
    d2p = jnp.concatenate([edge_index_r2[1], ipad])

    part1 = _sca(xwp_p, e1p, d1p)
    part2 = _sca(xwp_p, e2p, d2p)
    relf = _tc2(part1.reshape(NC, NP1, TWP),
                part2.reshape(NC, NP1, TWP))

    srci = jnp.concatenate([edge_index_r1[0], edge_index_r2[0]], axis=0)
    dsti = jnp.concatenate([edge_index_r1[1], edge_index_r2[1]], axis=0)
    sg, dg = _scb(relf, srci, dsti)

    wf = W_bl1.transpose(1, 0, 2).reshape(2 * D_OUT, D_OUT * 2 * D_OUT)
    sm = (jnp.arange(D_OUT * 2 * D_OUT)[:, None] // (2 * D_OUT)
          == jnp.arange(D_OUT)[None, :]).astype(jnp.float32)
    ecat = jnp.concatenate([e1, e2], axis=0)
    out = _tc3(sg, dg, ecat, wf, sm, b_bl1.reshape(1, D_OUT))
    return out.reshape(2, N_EDGES, D_OUT)
